# Initial kernel scaffold; baseline (speedup 1.0000x reference)
import jax
import jax.numpy as jnp
from jax.experimental import pallas as pl


def kernel(inputs, hidden_state, A, Wg, bg, Wu, bu, W, b):
    raise NotImplementedError("write your pallas kernel here")


# speedup vs baseline: 1.3814x; 1.3814x over previous
import jax
import jax.numpy as jnp
from jax.experimental import pallas as pl
from jax.experimental.pallas import tpu as pltpu


def _gru_kernel(xall_ref, h0_ref, a_ref, wr_ref, wu_ref, wc_ref, wy_ref,
                bgr_ref, bgu_ref, buc_ref, by_ref, y_ref, h_ref):
    A = a_ref[...]
    h = h0_ref[...]
    S = xall_ref.shape[0]
    K = wr_ref.shape[0]
    for t in range(S):
        x = xall_ref[t]
        cur = jnp.concatenate([x, h], axis=1)
        rpre = bgr_ref[...]
        upre = bgu_ref[...]
        for k in range(K):
            rpre = rpre + jnp.dot(cur, wr_ref[k],
                                  preferred_element_type=jnp.float32)
            upre = upre + jnp.dot(cur, wu_ref[k],
                                  preferred_element_type=jnp.float32)
            if k < K - 1:
                cur = jnp.dot(A, cur, preferred_element_type=jnp.float32)
        r = jax.nn.sigmoid(rpre)
        u = jax.nn.sigmoid(upre)
        cur = jnp.concatenate([x, r * h], axis=1)
        cpre = buc_ref[...]
        for k in range(K):
            cpre = cpre + jnp.dot(cur, wc_ref[k],
                                  preferred_element_type=jnp.float32)
            if k < K - 1:
                cur = jnp.dot(A, cur, preferred_element_type=jnp.float32)
        cy = jnp.tanh(cpre)
        h = u * h + (1.0 - u) * cy
    y_ref[...] = jax.nn.sigmoid(
        jnp.dot(h, wy_ref[...], preferred_element_type=jnp.float32)
        + by_ref[...])
    h_ref[...] = h


def kernel(inputs, hidden_state, A, Wg, bg, Wu, bu, W, b):
    B, S, N, F = inputs.shape
    K = Wg.shape[0]
    BF = B * F

    eye = jnp.eye(B, dtype=jnp.float32)

    def blockdiag(m):
        return jnp.kron(eye, m)

    xall = inputs.transpose(1, 2, 0, 3).reshape(S, N, BF)
    h0 = hidden_state.transpose(1, 0, 2).reshape(N, BF)

    def split_w(wk, cols):
        top = blockdiag(wk[:F, cols])
        bot = blockdiag(wk[F:, cols])
        return jnp.concatenate([top, bot], axis=0)

    wr = jnp.stack([split_w(Wg[k], slice(0, F)) for k in range(K)])
    wu = jnp.stack([split_w(Wg[k], slice(F, 2 * F)) for k in range(K)])
    wc = jnp.stack([split_w(Wu[k], slice(0, F)) for k in range(K)])
    wy = blockdiag(W)

    bgr = jnp.tile(bg[:F], B).reshape(1, BF)
    bgu = jnp.tile(bg[F:], B).reshape(1, BF)
    buc = jnp.tile(bu, B).reshape(1, BF)
    by = jnp.tile(b, B).reshape(1, BF)

    y, h = pl.pallas_call(
        _gru_kernel,
        out_shape=(
            jax.ShapeDtypeStruct((N, BF), jnp.float32),
            jax.ShapeDtypeStruct((N, BF), jnp.float32),
        ),
    )(xall, h0, A, wr, wu, wc, wy, bgr, bgu, buc, by)

    yt = y.reshape(N, B, F).transpose(1, 0, 2)
    hy = h.reshape(N, B, F).transpose(1, 0, 2)
    return (yt, hy)
